# baseline (speedup 1.0000x reference)
import functools

import jax
import jax.numpy as jnp
from jax import lax
from jax.experimental import pallas as pl
from jax.experimental.pallas import tpu as pltpu
from jax.experimental.pallas import tpu_sc as plsc

BATCH = 1024
SEQ = 20
TOK = BATCH * SEQ
EMBED = 32
HIDDEN = 64
OUT = 1000
VDIM = 1000000
PACK = 4
GROW = 128
VROWS = 253952
RB = 8192
NDB = VROWS // RB

NUM_CORES = 2
NUM_SUBCORES = 16
NW = NUM_CORES * NUM_SUBCORES
TPW = TOK // NW
CHUNK = 128
NCHUNK = TPW // CHUNK

_sc_mesh = plsc.VectorSubcoreMesh(core_axis_name="c", subcore_axis_name="s")


def _detile_body(x0_ref, x1_ref, x2_ref, x3_ref, eye_ref, out_ref):
    x = jnp.concatenate(
        [x0_ref[...], x1_ref[...], x2_ref[...], x3_ref[...]], axis=0)
    out_ref[...] = lax.dot_general(x, eye_ref[...], (((0,), (0,)), ((), ())),
                                   preferred_element_type=jnp.float32)


def _detile(vt_t):
    return pl.pallas_call(
        _detile_body,
        grid=(NDB,),
        in_specs=[
            pl.BlockSpec(
                (EMBED, RB),
                lambda i, q=q: (0, jnp.minimum(i + q * NDB, VDIM // RB)))
            for q in range(PACK)
        ] + [pl.BlockSpec((GROW, GROW), lambda i: (0, 0))],
        out_specs=pl.BlockSpec((RB, GROW), lambda i: (i, 0)),
        out_shape=jax.ShapeDtypeStruct((VROWS, GROW), jnp.float32),
        compiler_params=pltpu.CompilerParams(
            dimension_semantics=("parallel",)),
    )(vt_t, vt_t, vt_t, vt_t, jnp.eye(GROW, dtype=jnp.float32))


@functools.partial(
    pl.kernel,
    mesh=_sc_mesh,
    out_type=jax.ShapeDtypeStruct((TOK, GROW), jnp.float32),
    scratch_types=(
        [pltpu.VMEM((1, CHUNK), jnp.int32) for _ in range(NCHUNK)]
        + [pltpu.VMEM((TPW, GROW), jnp.float32), pltpu.SemaphoreType.DMA]
    ),
)
def _sc_gather(vid_hbm, vtab_hbm, out_hbm, *rest):
    idxs, rows, sem = list(rest[:NCHUNK]), rest[NCHUNK], rest[NCHUNK + 1]
    wid = lax.axis_index("s") * NUM_CORES + lax.axis_index("c")
    base = wid * TPW
    for j in range(NCHUNK):
        pltpu.sync_copy(vid_hbm.at[wid * NCHUNK + j], idxs[j])
    copies = []
    for j in range(NCHUNK):
        copies.append(pltpu.async_copy(
            vtab_hbm.at[idxs[j].at[0]],
            rows.at[pl.ds(j * CHUNK, CHUNK), :],
            sem))
    for c in copies:
        c.wait()
    pltpu.sync_copy(rows, out_hbm.at[pl.ds(base, TPW), :])


LB = 4


def _mlp_body(rows_ref, vq_ref, pid_ref, pt_ref, w1q_ref, w1p_ref, b1_ref,
              w3_ref, b3_ref, out_ref):
    for s in range(LB):
        vq = vq_ref[s]
        pid = pid_ref[s]
        rows = rows_ref[pl.ds(s * BATCH, BATCH), :]
        h = b1_ref[...]
        for q in range(PACK):
            hq = lax.dot_general(w1q_ref[q], rows, (((0,), (1,)), ((), ())),
                                 preferred_element_type=jnp.float32)
            mq = jnp.where(vq == q, 1.0, 0.0)
            h = h + hq * mq
        row_ids = lax.broadcasted_iota(jnp.int32, (OUT, BATCH), 0)
        onehot = jnp.where(row_ids == pid, 1.0, 0.0).astype(jnp.float32)
        pe = lax.dot_general(pt_ref[...], onehot, (((1,), (0,)), ((), ())),
                             preferred_element_type=jnp.float32)
        h = h + lax.dot_general(w1p_ref[...], pe, (((0,), (0,)), ((), ())),
                                preferred_element_type=jnp.float32)
        h = jnp.maximum(h, 0.0)
        logits = lax.dot_general(w3_ref[...], h, (((0,), (0,)), ((), ())),
                                 preferred_element_type=jnp.float32)
        logits = logits + b3_ref[...]
        m = jnp.max(logits, axis=0, keepdims=True)
        e = jnp.exp(logits - m)
        lse = jnp.log(jnp.sum(e, axis=0, keepdims=True)) + m
        out_ref[s] = logits - lse


def _mlp(rows, vq3, pids3, pt_t, w1q, w1p, b1c, w3, b3c):
    return pl.pallas_call(
        _mlp_body,
        grid=(SEQ // LB,),
        in_specs=[
            pl.BlockSpec((LB * BATCH, GROW), lambda i: (i, 0)),
            pl.BlockSpec((LB, 1, BATCH), lambda i: (i, 0, 0)),
            pl.BlockSpec((LB, 1, BATCH), lambda i: (i, 0, 0)),
            pl.BlockSpec((EMBED, OUT), lambda i: (0, 0)),
            pl.BlockSpec((PACK, GROW, HIDDEN), lambda i: (0, 0, 0)),
            pl.BlockSpec((EMBED, HIDDEN), lambda i: (0, 0)),
            pl.BlockSpec((HIDDEN, 1), lambda i: (0, 0)),
            pl.BlockSpec((HIDDEN, OUT), lambda i: (0, 0)),
            pl.BlockSpec((OUT, 1), lambda i: (0, 0)),
        ],
        out_specs=pl.BlockSpec((LB, OUT, BATCH), lambda i: (i, 0, 0)),
        out_shape=jax.ShapeDtypeStruct((SEQ, OUT, BATCH), jnp.float32),
        compiler_params=pltpu.CompilerParams(
            dimension_semantics=("parallel",)),
    )(rows, vq3, pids3, pt_t, w1q, w1p, b1c, w3, b3c)


def kernel(vessel_ids, port_ids, vessel_table, port_table, W1, b1, W3, b3):
    vids_tau = vessel_ids.T.reshape(TOK).astype(jnp.int32)
    vrow = (vids_tau % VROWS).reshape(NW * NCHUNK, 1, CHUNK)
    vq3 = (vids_tau // VROWS).reshape(SEQ, 1, BATCH)
    pids3 = port_ids.T.reshape(SEQ, 1, BATCH).astype(jnp.int32)
    pt_t = port_table.T
    w1v = W1[:EMBED]
    w1q = jnp.stack([
        jnp.pad(w1v, ((q * EMBED, GROW - (q + 1) * EMBED), (0, 0)))
        for q in range(PACK)])
    vt4 = _detile(vessel_table.T)
    rows = _sc_gather(vrow, vt4)
    out_t = _mlp(rows, vq3, pids3, pt_t, w1q, W1[EMBED:],
                 b1.reshape(HIDDEN, 1), W3, b3.reshape(OUT, 1))
    return out_t.transpose(2, 0, 1)


# speedup vs baseline: 1.0114x; 1.0114x over previous
import functools

import jax
import jax.numpy as jnp
from jax import lax
from jax.experimental import pallas as pl
from jax.experimental.pallas import tpu as pltpu
from jax.experimental.pallas import tpu_sc as plsc

BATCH = 1024
SEQ = 20
TOK = BATCH * SEQ
EMBED = 32
HIDDEN = 64
OUT = 1000
VDIM = 1000000
PACK = 4
GROW = 128
VROWS = 253952
RB = 8192
NDB = VROWS // RB

NUM_CORES = 2
NUM_SUBCORES = 16
NW = NUM_CORES * NUM_SUBCORES
TPW = TOK // NW
CHUNK = 128
NCHUNK = TPW // CHUNK

_sc_mesh = plsc.VectorSubcoreMesh(core_axis_name="c", subcore_axis_name="s")


def _detile_body(x0_ref, x1_ref, x2_ref, x3_ref, eye_ref, out_ref):
    x = jnp.concatenate(
        [x0_ref[...], x1_ref[...], x2_ref[...], x3_ref[...]], axis=0)
    out_ref[...] = lax.dot_general(x, eye_ref[...], (((0,), (0,)), ((), ())),
                                   preferred_element_type=jnp.float32)


def _detile(vt_t):
    return pl.pallas_call(
        _detile_body,
        grid=(NDB,),
        in_specs=[
            pl.BlockSpec(
                (EMBED, RB),
                lambda i, q=q: (0, jnp.minimum(i + q * NDB, VDIM // RB)))
            for q in range(PACK)
        ] + [pl.BlockSpec((GROW, GROW), lambda i: (0, 0))],
        out_specs=pl.BlockSpec((RB, GROW), lambda i: (i, 0)),
        out_shape=jax.ShapeDtypeStruct((VROWS, GROW), jnp.float32),
        compiler_params=pltpu.CompilerParams(
            dimension_semantics=("parallel",)),
    )(vt_t, vt_t, vt_t, vt_t, jnp.eye(GROW, dtype=jnp.float32))


@functools.partial(
    pl.kernel,
    mesh=_sc_mesh,
    out_type=jax.ShapeDtypeStruct((TOK, GROW), jnp.float32),
    scratch_types=(
        [pltpu.VMEM((1, CHUNK), jnp.int32) for _ in range(NCHUNK)]
        + [pltpu.VMEM((TPW, GROW), jnp.float32), pltpu.SemaphoreType.DMA]
    ),
)
def _sc_gather(vid_hbm, vtab_hbm, out_hbm, *rest):
    idxs, rows, sem = list(rest[:NCHUNK]), rest[NCHUNK], rest[NCHUNK + 1]
    wid = lax.axis_index("s") * NUM_CORES + lax.axis_index("c")
    base = wid * TPW
    for j in range(NCHUNK):
        pltpu.sync_copy(vid_hbm.at[wid * NCHUNK + j], idxs[j])
    copies = []
    for j in range(NCHUNK):
        copies.append(pltpu.async_copy(
            vtab_hbm.at[idxs[j].at[0]],
            rows.at[pl.ds(j * CHUNK, CHUNK), :],
            sem))
    for c in copies:
        c.wait()
    pltpu.sync_copy(rows, out_hbm.at[pl.ds(base, TPW), :])


LB = 1


def _mlp_body(rows_ref, vq_ref, pid_ref, pt_ref, w1q_ref, w1p_ref, b1_ref,
              w3_ref, b3_ref, out_ref):
    for s in range(LB):
        vq = vq_ref[s]
        pid = pid_ref[s]
        rows = rows_ref[pl.ds(s * BATCH, BATCH), :]
        h = b1_ref[...]
        for q in range(PACK):
            hq = lax.dot_general(w1q_ref[q], rows, (((0,), (1,)), ((), ())),
                                 preferred_element_type=jnp.float32)
            mq = jnp.where(vq == q, 1.0, 0.0)
            h = h + hq * mq
        row_ids = lax.broadcasted_iota(jnp.int32, (OUT, BATCH), 0)
        onehot = jnp.where(row_ids == pid, 1.0, 0.0).astype(jnp.float32)
        pe = lax.dot_general(pt_ref[...], onehot, (((1,), (0,)), ((), ())),
                             preferred_element_type=jnp.float32)
        h = h + lax.dot_general(w1p_ref[...], pe, (((0,), (0,)), ((), ())),
                                preferred_element_type=jnp.float32)
        h = jnp.maximum(h, 0.0)
        logits = lax.dot_general(w3_ref[...], h, (((0,), (0,)), ((), ())),
                                 preferred_element_type=jnp.float32)
        logits = logits + b3_ref[...]
        m = jnp.max(logits, axis=0, keepdims=True)
        e = jnp.exp(logits - m)
        lse = jnp.log(jnp.sum(e, axis=0, keepdims=True)) + m
        out_ref[s] = logits - lse


def _mlp(rows, vq3, pids3, pt_t, w1q, w1p, b1c, w3, b3c):
    return pl.pallas_call(
        _mlp_body,
        grid=(SEQ // LB,),
        in_specs=[
            pl.BlockSpec((LB * BATCH, GROW), lambda i: (i, 0)),
            pl.BlockSpec((LB, 1, BATCH), lambda i: (i, 0, 0)),
            pl.BlockSpec((LB, 1, BATCH), lambda i: (i, 0, 0)),
            pl.BlockSpec((EMBED, OUT), lambda i: (0, 0)),
            pl.BlockSpec((PACK, GROW, HIDDEN), lambda i: (0, 0, 0)),
            pl.BlockSpec((EMBED, HIDDEN), lambda i: (0, 0)),
            pl.BlockSpec((HIDDEN, 1), lambda i: (0, 0)),
            pl.BlockSpec((HIDDEN, OUT), lambda i: (0, 0)),
            pl.BlockSpec((OUT, 1), lambda i: (0, 0)),
        ],
        out_specs=pl.BlockSpec((LB, OUT, BATCH), lambda i: (i, 0, 0)),
        out_shape=jax.ShapeDtypeStruct((SEQ, OUT, BATCH), jnp.float32),
        compiler_params=pltpu.CompilerParams(
            dimension_semantics=("parallel",)),
    )(rows, vq3, pids3, pt_t, w1q, w1p, b1c, w3, b3c)


def kernel(vessel_ids, port_ids, vessel_table, port_table, W1, b1, W3, b3):
    vids_tau = vessel_ids.T.reshape(TOK).astype(jnp.int32)
    vrow = (vids_tau % VROWS).reshape(NW * NCHUNK, 1, CHUNK)
    vq3 = (vids_tau // VROWS).reshape(SEQ, 1, BATCH)
    pids3 = port_ids.T.reshape(SEQ, 1, BATCH).astype(jnp.int32)
    pt_t = port_table.T
    w1v = W1[:EMBED]
    w1q = jnp.stack([
        jnp.pad(w1v, ((q * EMBED, GROW - (q + 1) * EMBED), (0, 0)))
        for q in range(PACK)])
    vt4 = _detile(vessel_table.T)
    rows = _sc_gather(vrow, vt4)
    out_t = _mlp(rows, vq3, pids3, pt_t, w1q, W1[EMBED:],
                 b1.reshape(HIDDEN, 1), W3, b3.reshape(OUT, 1))
    return out_t.transpose(2, 0, 1)
